# Initial kernel scaffold; baseline (speedup 1.0000x reference)
import jax
import jax.numpy as jnp
from jax.experimental import pallas as pl


def kernel(x, edge_index, W1, b1, W2, b2):
    raise NotImplementedError("write your pallas kernel here")


# speedup vs baseline: 29.9901x; 29.9901x over previous
import functools
import jax
import jax.numpy as jnp
from jax import lax
from jax.experimental import pallas as pl
from jax.experimental.pallas import tpu as pltpu
from jax.experimental.pallas import tpu_sc as plsc

N = 10000
E = 320000
D = 128
H = 16
OUT = 7

NC = 2
NS = 16
NW = NC * NS
EW = E // NW
C = 80
K = EW // C
NPAD = 10240
RS1 = NPAD // NS
RS2 = NPAD // NS

_sc_mesh = plsc.VectorSubcoreMesh(
    core_axis_name="c", subcore_axis_name="s", num_cores=NC, num_subcores=NS)
_sc_params = pltpu.CompilerParams(use_tc_tiling_on_sc=False)


@functools.partial(
    pl.kernel,
    out_type=jax.ShapeDtypeStruct((NC, NPAD), jnp.float32),
    mesh=_sc_mesh,
    compiler_params=_sc_params,
    scratch_types=[
        pltpu.VMEM((K, C), jnp.int32),
        pltpu.VMEM((C,), jnp.float32),
        pltpu.VMEM_SHARED((NPAD,), jnp.float32),
    ],
)
def _deg_kernel(dsts_hbm, ones_hbm, zeros_hbm, out_hbm, dst_v, ones_v, acc_sh):
    c = lax.axis_index("c")
    s = lax.axis_index("s")
    wid = c * NS + s
    pltpu.sync_copy(dsts_hbm.at[wid], dst_v)
    pltpu.sync_copy(ones_hbm, ones_v)
    pltpu.sync_copy(zeros_hbm.at[pl.ds(s * RS1, RS1)],
                    acc_sh.at[pl.ds(s * RS1, RS1)])
    plsc.subcore_barrier()

    def step(j, carry):
        pltpu.sync_copy(ones_v, acc_sh.at[dst_v.at[j]], add=True)
        return carry

    lax.fori_loop(0, K, step, 0)
    plsc.subcore_barrier()
    pltpu.sync_copy(acc_sh.at[pl.ds(s * RS1, RS1)],
                    out_hbm.at[c, pl.ds(s * RS1, RS1)])


@functools.partial(
    pl.kernel,
    out_type=jax.ShapeDtypeStruct((NC, NPAD, H), jnp.float32),
    mesh=_sc_mesh,
    compiler_params=_sc_params,
    scratch_types=[
        pltpu.VMEM((K, C), jnp.int32),
        pltpu.VMEM((K, C), jnp.int32),
        pltpu.VMEM((C, H), jnp.float32),
        pltpu.VMEM_SHARED((NPAD, H), jnp.float32),
        pltpu.SemaphoreType.DMA,
    ],
)
def _agg_kernel(table_hbm, srcs_hbm, dsts_hbm, zeros_hbm, out_hbm,
                src_v, dst_v, rows_v, acc_sh, sem):
    c = lax.axis_index("c")
    s = lax.axis_index("s")
    wid = c * NS + s
    pltpu.sync_copy(srcs_hbm.at[wid], src_v)
    pltpu.sync_copy(dsts_hbm.at[wid], dst_v)
    pltpu.sync_copy(zeros_hbm.at[pl.ds(s * RS2, RS2)],
                    acc_sh.at[pl.ds(s * RS2, RS2)])
    plsc.subcore_barrier()

    def step(j, carry):
        pltpu.async_copy(table_hbm.at[src_v.at[j]], rows_v, sem).wait()
        pltpu.sync_copy(rows_v, acc_sh.at[dst_v.at[j]], add=True)
        return carry

    lax.fori_loop(0, K, step, 0)
    plsc.subcore_barrier()
    pltpu.sync_copy(acc_sh.at[pl.ds(s * RS2, RS2)],
                    out_hbm.at[c, pl.ds(s * RS2, RS2)])


def _mm1_body(x_ref, w1_ref, degp_ref, h1s_ref, dinv_ref):
    deg = degp_ref[0, :N] + degp_ref[1, :N] + 1.0
    dinv = lax.rsqrt(deg)[:, None]
    h1s_ref[...] = jnp.dot(x_ref[...], w1_ref[...],
                           preferred_element_type=jnp.float32) * dinv
    dinv_ref[...] = dinv


def _mid_body(p_ref, h1s_ref, dinv_ref, b1_ref, w2_ref, h2s_ref):
    dinv = dinv_ref[...]
    z = (p_ref[0, :N] + p_ref[1, :N] + h1s_ref[...]) * dinv + b1_ref[...]
    r = jnp.maximum(z, 0.0)
    h2s_ref[...] = jnp.dot(r, w2_ref[...],
                           preferred_element_type=jnp.float32) * dinv


def _out_body(q_ref, h2s_ref, dinv_ref, b2_ref, o_ref):
    o = (q_ref[0, :N] + q_ref[1, :N] + h2s_ref[...]) * dinv_ref[...] + b2_ref[...]
    col = lax.broadcasted_iota(jnp.int32, (N, H), 1)
    om = jnp.where(col < OUT, o, -1e30)
    m = jnp.max(om, axis=1, keepdims=True)
    lse = jnp.log(jnp.sum(jnp.exp(om - m), axis=1, keepdims=True)) + m
    o_ref[...] = om - lse


def kernel(x, edge_index, W1, b1, W2, b2):
    ei = edge_index.astype(jnp.int32)
    srcs = ei[0].reshape(NW, K, C)
    dsts = ei[1].reshape(NW, K, C)
    zeros1 = jnp.zeros((NPAD,), jnp.float32)
    zeros2 = jnp.zeros((NPAD, H), jnp.float32)
    ones_c = jnp.ones((C,), jnp.float32)
    w2p = jnp.zeros((H, H), jnp.float32).at[:, :OUT].set(W2)
    b1r = b1.reshape(1, H)
    b2r = jnp.zeros((1, H), jnp.float32).at[0, :OUT].set(b2)

    degp = _deg_kernel(dsts, ones_c, zeros1)

    h1s, dinv = pl.pallas_call(
        _mm1_body,
        out_shape=[jax.ShapeDtypeStruct((N, H), jnp.float32),
                   jax.ShapeDtypeStruct((N, 1), jnp.float32)],
    )(x, W1, degp)

    p = _agg_kernel(h1s, srcs, dsts, zeros2)

    h2s = pl.pallas_call(
        _mid_body,
        out_shape=jax.ShapeDtypeStruct((N, H), jnp.float32),
    )(p, h1s, dinv, b1r, w2p)

    q = _agg_kernel(h2s, srcs, dsts, zeros2)

    o = pl.pallas_call(
        _out_body,
        out_shape=jax.ShapeDtypeStruct((N, H), jnp.float32),
    )(q, h2s, dinv, b2r)

    return o[:, :OUT]


# speedup vs baseline: 57.9152x; 1.9311x over previous
import functools
import jax
import jax.numpy as jnp
from jax import lax
from jax.experimental import pallas as pl
from jax.experimental.pallas import tpu as pltpu
from jax.experimental.pallas import tpu_sc as plsc

N = 10000
E = 320000
D = 128
H = 16
OUT = 7

NC = 2
NS = 16
NW = NC * NS
EW = E // NW
C = 80
K = EW // C
NPAD = 10240
RS1 = NPAD // NS
RS2 = NPAD // NS
NB = 5
G = K // NB
DEPTH1 = 8

_sc_mesh = plsc.VectorSubcoreMesh(
    core_axis_name="c", subcore_axis_name="s", num_cores=NC, num_subcores=NS)
_sc_params = pltpu.CompilerParams(use_tc_tiling_on_sc=False)


@functools.partial(
    pl.kernel,
    out_type=jax.ShapeDtypeStruct((NC, NPAD), jnp.float32),
    mesh=_sc_mesh,
    compiler_params=_sc_params,
    scratch_types=[
        pltpu.VMEM((K, C), jnp.int32),
        pltpu.VMEM((C,), jnp.float32),
        pltpu.VMEM_SHARED((NPAD,), jnp.float32),
        pltpu.SemaphoreType.DMA,
    ],
)
def _deg_kernel(dsts_hbm, ones_hbm, zeros_hbm, out_hbm, dst_v, ones_v, acc_sh,
                ssem):
    c = lax.axis_index("c")
    s = lax.axis_index("s")
    wid = c * NS + s
    pltpu.sync_copy(dsts_hbm.at[wid], dst_v)
    pltpu.sync_copy(ones_hbm, ones_v)
    pltpu.sync_copy(zeros_hbm.at[pl.ds(s * RS1, RS1)],
                    acc_sh.at[pl.ds(s * RS1, RS1)])
    plsc.subcore_barrier()

    def step(j, carry):
        pltpu.async_copy(ones_v, acc_sh.at[dst_v.at[j]], ssem, add=True)

        @pl.when(j >= DEPTH1)
        def _():
            pltpu.make_async_copy(ones_v, acc_sh.at[dst_v.at[j]], ssem).wait()

        return carry

    lax.fori_loop(0, K, step, 0)
    for _ in range(DEPTH1):
        pltpu.make_async_copy(ones_v, acc_sh.at[dst_v.at[0]], ssem).wait()
    plsc.subcore_barrier()
    pltpu.sync_copy(acc_sh.at[pl.ds(s * RS1, RS1)],
                    out_hbm.at[c, pl.ds(s * RS1, RS1)])


@functools.partial(
    pl.kernel,
    out_type=jax.ShapeDtypeStruct((NC, NPAD, H), jnp.float32),
    mesh=_sc_mesh,
    compiler_params=_sc_params,
    scratch_types=[
        pltpu.VMEM((K, C), jnp.int32),
        pltpu.VMEM((K, C), jnp.int32),
        pltpu.VMEM((NB, C, H), jnp.float32),
        pltpu.VMEM_SHARED((NPAD, H), jnp.float32),
        pltpu.SemaphoreType.DMA((NB,)),
        pltpu.SemaphoreType.DMA((NB,)),
    ],
)
def _agg_kernel(table_hbm, srcs_hbm, dsts_hbm, zeros_hbm, out_hbm,
                src_v, dst_v, rows_v, acc_sh, gsem, ssem):
    c = lax.axis_index("c")
    s = lax.axis_index("s")
    wid = c * NS + s
    pltpu.sync_copy(srcs_hbm.at[wid], src_v)
    pltpu.sync_copy(dsts_hbm.at[wid], dst_v)
    pltpu.sync_copy(zeros_hbm.at[pl.ds(s * RS2, RS2)],
                    acc_sh.at[pl.ds(s * RS2, RS2)])
    plsc.subcore_barrier()

    for b in range(NB):
        pltpu.async_copy(table_hbm.at[src_v.at[b]], rows_v.at[b], gsem.at[b])

    def outer(g, carry):
        base = g * NB
        for b in range(NB):
            j = base + b
            pltpu.make_async_copy(table_hbm.at[src_v.at[j]], rows_v.at[b],
                                  gsem.at[b]).wait()
            pltpu.async_copy(rows_v.at[b], acc_sh.at[dst_v.at[j]],
                             ssem.at[b], add=True)

        @pl.when(g < G - 1)
        def _():
            for b in range(NB):
                j = base + b
                pltpu.make_async_copy(rows_v.at[b], acc_sh.at[dst_v.at[j]],
                                      ssem.at[b]).wait()
                pltpu.async_copy(table_hbm.at[src_v.at[j + NB]],
                                 rows_v.at[b], gsem.at[b])

        return carry

    lax.fori_loop(0, G, outer, 0)
    for b in range(NB):
        pltpu.make_async_copy(rows_v.at[b], acc_sh.at[dst_v.at[b]],
                              ssem.at[b]).wait()
    plsc.subcore_barrier()
    pltpu.sync_copy(acc_sh.at[pl.ds(s * RS2, RS2)],
                    out_hbm.at[c, pl.ds(s * RS2, RS2)])


def _mm1_body(x_ref, w1_ref, degp_ref, h1s_ref, dinv_ref):
    deg = degp_ref[0, :N] + degp_ref[1, :N] + 1.0
    dinv = lax.rsqrt(deg)[:, None]
    h1s_ref[...] = jnp.dot(x_ref[...], w1_ref[...],
                           preferred_element_type=jnp.float32) * dinv
    dinv_ref[...] = dinv


def _mid_body(p_ref, h1s_ref, dinv_ref, b1_ref, w2_ref, h2s_ref):
    dinv = dinv_ref[...]
    z = (p_ref[0, :N] + p_ref[1, :N] + h1s_ref[...]) * dinv + b1_ref[...]
    r = jnp.maximum(z, 0.0)
    h2s_ref[...] = jnp.dot(r, w2_ref[...],
                           preferred_element_type=jnp.float32) * dinv


def _out_body(q_ref, h2s_ref, dinv_ref, b2_ref, o_ref):
    o = (q_ref[0, :N] + q_ref[1, :N] + h2s_ref[...]) * dinv_ref[...] + b2_ref[...]
    col = lax.broadcasted_iota(jnp.int32, (N, H), 1)
    om = jnp.where(col < OUT, o, -1e30)
    m = jnp.max(om, axis=1, keepdims=True)
    lse = jnp.log(jnp.sum(jnp.exp(om - m), axis=1, keepdims=True)) + m
    o_ref[...] = om - lse


def kernel(x, edge_index, W1, b1, W2, b2):
    ei = edge_index.astype(jnp.int32)
    srcs = ei[0].reshape(NW, K, C)
    dsts = ei[1].reshape(NW, K, C)
    zeros1 = jnp.zeros((NPAD,), jnp.float32)
    zeros2 = jnp.zeros((NPAD, H), jnp.float32)
    ones_c = jnp.ones((C,), jnp.float32)
    w2p = jnp.zeros((H, H), jnp.float32).at[:, :OUT].set(W2)
    b1r = b1.reshape(1, H)
    b2r = jnp.zeros((1, H), jnp.float32).at[0, :OUT].set(b2)

    degp = _deg_kernel(dsts, ones_c, zeros1)

    h1s, dinv = pl.pallas_call(
        _mm1_body,
        out_shape=[jax.ShapeDtypeStruct((N, H), jnp.float32),
                   jax.ShapeDtypeStruct((N, 1), jnp.float32)],
    )(x, W1, degp)

    p = _agg_kernel(h1s, srcs, dsts, zeros2)

    h2s = pl.pallas_call(
        _mid_body,
        out_shape=jax.ShapeDtypeStruct((N, H), jnp.float32),
    )(p, h1s, dinv, b1r, w2p)

    q = _agg_kernel(h2s, srcs, dsts, zeros2)

    o = pl.pallas_call(
        _out_body,
        out_shape=jax.ShapeDtypeStruct((N, H), jnp.float32),
    )(q, h2s, dinv, b2r)

    return o[:, :OUT]


# speedup vs baseline: 59.6132x; 1.0293x over previous
import functools
import jax
import jax.numpy as jnp
from jax import lax
from jax.experimental import pallas as pl
from jax.experimental.pallas import tpu as pltpu
from jax.experimental.pallas import tpu_sc as plsc

N = 10000
E = 320000
D = 128
H = 16
OUT = 7

NC = 2
NS = 16
NW = NC * NS
EPAD = 327680
EW = EPAD // NW
C = 128
K = EW // C
NPAD = 10240
RS1 = NPAD // NS
RS2 = NPAD // NS
NB = 4
G = K // NB
DEPTH1 = 8
assert K % NB == 0

_sc_mesh = plsc.VectorSubcoreMesh(
    core_axis_name="c", subcore_axis_name="s", num_cores=NC, num_subcores=NS)
_sc_params = pltpu.CompilerParams(use_tc_tiling_on_sc=False)


@functools.partial(
    pl.kernel,
    out_type=jax.ShapeDtypeStruct((NC, NPAD), jnp.float32),
    mesh=_sc_mesh,
    compiler_params=_sc_params,
    scratch_types=[
        pltpu.VMEM((K, C), jnp.int32),
        pltpu.VMEM((C,), jnp.float32),
        pltpu.VMEM_SHARED((NPAD,), jnp.float32),
        pltpu.SemaphoreType.DMA,
    ],
)
def _deg_kernel(dsts_hbm, ones_hbm, zeros_hbm, out_hbm, dst_v, ones_v, acc_sh,
                ssem):
    c = lax.axis_index("c")
    s = lax.axis_index("s")
    wid = c * NS + s
    pltpu.sync_copy(dsts_hbm.at[wid], dst_v)
    pltpu.sync_copy(ones_hbm, ones_v)
    pltpu.sync_copy(zeros_hbm.at[pl.ds(s * RS1, RS1)],
                    acc_sh.at[pl.ds(s * RS1, RS1)])
    plsc.subcore_barrier()

    def step(j, carry):
        pltpu.async_copy(ones_v, acc_sh.at[dst_v.at[j]], ssem, add=True)

        @pl.when(j >= DEPTH1)
        def _():
            pltpu.make_async_copy(ones_v, acc_sh.at[dst_v.at[j]], ssem).wait()

        return carry

    lax.fori_loop(0, K, step, 0)
    for _ in range(DEPTH1):
        pltpu.make_async_copy(ones_v, acc_sh.at[dst_v.at[0]], ssem).wait()
    plsc.subcore_barrier()
    pltpu.sync_copy(acc_sh.at[pl.ds(s * RS1, RS1)],
                    out_hbm.at[c, pl.ds(s * RS1, RS1)])


@functools.partial(
    pl.kernel,
    out_type=jax.ShapeDtypeStruct((NC, NPAD, H), jnp.float32),
    mesh=_sc_mesh,
    compiler_params=_sc_params,
    scratch_types=[
        pltpu.VMEM((K, C), jnp.int32),
        pltpu.VMEM((K, C), jnp.int32),
        pltpu.VMEM((NB, C, H), jnp.float32),
        pltpu.VMEM_SHARED((NPAD, H), jnp.float32),
        pltpu.SemaphoreType.DMA((NB,)),
        pltpu.SemaphoreType.DMA((NB,)),
    ],
)
def _agg_kernel(table_hbm, srcs_hbm, dsts_hbm, zeros_hbm, out_hbm,
                src_v, dst_v, rows_v, acc_sh, gsem, ssem):
    c = lax.axis_index("c")
    s = lax.axis_index("s")
    wid = c * NS + s
    pltpu.sync_copy(srcs_hbm.at[wid], src_v)
    pltpu.sync_copy(dsts_hbm.at[wid], dst_v)
    pltpu.sync_copy(zeros_hbm.at[pl.ds(s * RS2, RS2)],
                    acc_sh.at[pl.ds(s * RS2, RS2)])
    plsc.subcore_barrier()

    for b in range(NB):
        pltpu.async_copy(table_hbm.at[src_v.at[b]], rows_v.at[b], gsem.at[b])

    def outer(g, carry):
        base = g * NB
        for b in range(NB):
            j = base + b
            pltpu.make_async_copy(table_hbm.at[src_v.at[j]], rows_v.at[b],
                                  gsem.at[b]).wait()
            pltpu.async_copy(rows_v.at[b], acc_sh.at[dst_v.at[j]],
                             ssem.at[b], add=True)

        @pl.when(g < G - 1)
        def _():
            for b in range(NB):
                j = base + b
                pltpu.make_async_copy(rows_v.at[b], acc_sh.at[dst_v.at[j]],
                                      ssem.at[b]).wait()
                pltpu.async_copy(table_hbm.at[src_v.at[j + NB]],
                                 rows_v.at[b], gsem.at[b])

        return carry

    lax.fori_loop(0, G, outer, 0)
    for b in range(NB):
        pltpu.make_async_copy(rows_v.at[b], acc_sh.at[dst_v.at[b]],
                              ssem.at[b]).wait()
    plsc.subcore_barrier()
    pltpu.sync_copy(acc_sh.at[pl.ds(s * RS2, RS2)],
                    out_hbm.at[c, pl.ds(s * RS2, RS2)])


def _mm1_body(x_ref, w1_ref, degp_ref, h1s_ref, dinv_ref):
    deg = degp_ref[0, :N] + degp_ref[1, :N] + 1.0
    dinv = lax.rsqrt(deg)[:, None]
    h1s_ref[...] = jnp.dot(x_ref[...], w1_ref[...],
                           preferred_element_type=jnp.float32) * dinv
    dinv_ref[...] = dinv


def _mid_body(p_ref, h1s_ref, dinv_ref, b1_ref, rs_ref):
    dinv = dinv_ref[...]
    z = (p_ref[0, :N] + p_ref[1, :N] + h1s_ref[...]) * dinv + b1_ref[...]
    rs_ref[...] = jnp.maximum(z, 0.0) * dinv


def _out_body(q_ref, rs_ref, dinv_ref, b2_ref, w2_ref, o_ref):
    t = (q_ref[0, :N] + q_ref[1, :N] + rs_ref[...]) * dinv_ref[...]
    o = jnp.dot(t, w2_ref[...], preferred_element_type=jnp.float32) + b2_ref[...]
    col = lax.broadcasted_iota(jnp.int32, (N, H), 1)
    om = jnp.where(col < OUT, o, -1e30)
    m = jnp.max(om, axis=1, keepdims=True)
    lse = jnp.log(jnp.sum(jnp.exp(om - m), axis=1, keepdims=True)) + m
    o_ref[...] = om - lse


def kernel(x, edge_index, W1, b1, W2, b2):
    ei = edge_index.astype(jnp.int32)
    npd = EPAD - E
    src_pad = (jnp.arange(npd, dtype=jnp.int32) * 97) % N
    dst_pad = N + (jnp.arange(npd, dtype=jnp.int32) % (NPAD - N))
    srcs = jnp.concatenate([ei[0], src_pad]).reshape(NW, K, C)
    dsts = jnp.concatenate([ei[1], dst_pad]).reshape(NW, K, C)
    zeros1 = jnp.zeros((NPAD,), jnp.float32)
    zeros2 = jnp.zeros((NPAD, H), jnp.float32)
    ones_c = jnp.ones((C,), jnp.float32)
    w2p = jnp.zeros((H, H), jnp.float32).at[:, :OUT].set(W2)
    b1r = b1.reshape(1, H)
    b2r = jnp.zeros((1, H), jnp.float32).at[0, :OUT].set(b2)

    degp = _deg_kernel(dsts, ones_c, zeros1)

    h1s, dinv = pl.pallas_call(
        _mm1_body,
        out_shape=[jax.ShapeDtypeStruct((N, H), jnp.float32),
                   jax.ShapeDtypeStruct((N, 1), jnp.float32)],
    )(x, W1, degp)

    p = _agg_kernel(h1s, srcs, dsts, zeros2)

    rs = pl.pallas_call(
        _mid_body,
        out_shape=jax.ShapeDtypeStruct((N, H), jnp.float32),
    )(p, h1s, dinv, b1r)

    q = _agg_kernel(rs, srcs, dsts, zeros2)

    o = pl.pallas_call(
        _out_body,
        out_shape=jax.ShapeDtypeStruct((N, H), jnp.float32),
    )(q, rs, dinv, b2r, w2p)

    return o[:, :OUT]


# speedup vs baseline: 64.6251x; 1.0841x over previous
import functools
import jax
import jax.numpy as jnp
from jax import lax
from jax.experimental import pallas as pl
from jax.experimental.pallas import tpu as pltpu
from jax.experimental.pallas import tpu_sc as plsc

N = 10000
E = 320000
D = 128
H = 16
OUT = 7

NC = 2
NS = 16
NW = NC * NS
EPAD = 327680
EW = EPAD // NW
C = 512
K = EW // C
NPAD = 10240
RS1 = NPAD // NS
RS2 = NPAD // NS
NB = 4
G = K // NB
DEPTH1 = 8
assert K % NB == 0

_sc_mesh = plsc.VectorSubcoreMesh(
    core_axis_name="c", subcore_axis_name="s", num_cores=NC, num_subcores=NS)
_sc_params = pltpu.CompilerParams(use_tc_tiling_on_sc=False)


@functools.partial(
    pl.kernel,
    out_type=jax.ShapeDtypeStruct((NC, NPAD), jnp.float32),
    mesh=_sc_mesh,
    compiler_params=_sc_params,
    scratch_types=[
        pltpu.VMEM((K, C), jnp.int32),
        pltpu.VMEM((C,), jnp.float32),
        pltpu.VMEM_SHARED((NPAD,), jnp.float32),
        pltpu.SemaphoreType.DMA,
    ],
)
def _deg_kernel(dsts_hbm, ones_hbm, zeros_hbm, out_hbm, dst_v, ones_v, acc_sh,
                ssem):
    c = lax.axis_index("c")
    s = lax.axis_index("s")
    wid = c * NS + s
    pltpu.sync_copy(dsts_hbm.at[wid], dst_v)
    pltpu.sync_copy(ones_hbm, ones_v)
    pltpu.sync_copy(zeros_hbm.at[pl.ds(s * RS1, RS1)],
                    acc_sh.at[pl.ds(s * RS1, RS1)])
    plsc.subcore_barrier()

    def step(j, carry):
        pltpu.async_copy(ones_v, acc_sh.at[dst_v.at[j]], ssem, add=True)

        @pl.when(j >= DEPTH1)
        def _():
            pltpu.make_async_copy(ones_v, acc_sh.at[dst_v.at[j]], ssem).wait()

        return carry

    lax.fori_loop(0, K, step, 0)
    for _ in range(DEPTH1):
        pltpu.make_async_copy(ones_v, acc_sh.at[dst_v.at[0]], ssem).wait()
    plsc.subcore_barrier()
    pltpu.sync_copy(acc_sh.at[pl.ds(s * RS1, RS1)],
                    out_hbm.at[c, pl.ds(s * RS1, RS1)])


@functools.partial(
    pl.kernel,
    out_type=jax.ShapeDtypeStruct((NC, NPAD, H), jnp.float32),
    mesh=_sc_mesh,
    compiler_params=_sc_params,
    scratch_types=[
        pltpu.VMEM((K, C), jnp.int32),
        pltpu.VMEM((K, C), jnp.int32),
        pltpu.VMEM((NB, C, H), jnp.float32),
        pltpu.VMEM_SHARED((NPAD, H), jnp.float32),
        pltpu.SemaphoreType.DMA((NB,)),
        pltpu.SemaphoreType.DMA((NB,)),
    ],
)
def _agg_kernel(table_hbm, srcs_hbm, dsts_hbm, zeros_hbm, out_hbm,
                src_v, dst_v, rows_v, acc_sh, gsem, ssem):
    c = lax.axis_index("c")
    s = lax.axis_index("s")
    wid = c * NS + s
    pltpu.sync_copy(srcs_hbm.at[wid], src_v)
    pltpu.sync_copy(dsts_hbm.at[wid], dst_v)
    pltpu.sync_copy(zeros_hbm.at[pl.ds(s * RS2, RS2)],
                    acc_sh.at[pl.ds(s * RS2, RS2)])
    plsc.subcore_barrier()

    for b in range(NB):
        pltpu.async_copy(table_hbm.at[src_v.at[b]], rows_v.at[b], gsem.at[b])

    def outer(g, carry):
        base = g * NB
        for b in range(NB):
            j = base + b
            pltpu.make_async_copy(table_hbm.at[src_v.at[j]], rows_v.at[b],
                                  gsem.at[b]).wait()
            pltpu.async_copy(rows_v.at[b], acc_sh.at[dst_v.at[j]],
                             ssem.at[b], add=True)

        @pl.when(g < G - 1)
        def _():
            for b in range(NB):
                j = base + b
                pltpu.make_async_copy(rows_v.at[b], acc_sh.at[dst_v.at[j]],
                                      ssem.at[b]).wait()
                pltpu.async_copy(table_hbm.at[src_v.at[j + NB]],
                                 rows_v.at[b], gsem.at[b])

        return carry

    lax.fori_loop(0, G, outer, 0)
    for b in range(NB):
        pltpu.make_async_copy(rows_v.at[b], acc_sh.at[dst_v.at[b]],
                              ssem.at[b]).wait()
    plsc.subcore_barrier()
    pltpu.sync_copy(acc_sh.at[pl.ds(s * RS2, RS2)],
                    out_hbm.at[c, pl.ds(s * RS2, RS2)])


def _mm1_body(x_ref, w1_ref, degp_ref, h1s_ref, dinv_ref):
    deg = degp_ref[0, :N] + degp_ref[1, :N] + 1.0
    dinv = lax.rsqrt(deg)[:, None]
    h1s_ref[...] = jnp.dot(x_ref[...], w1_ref[...],
                           preferred_element_type=jnp.float32) * dinv
    dinv_ref[...] = dinv


def _mid_body(p_ref, h1s_ref, dinv_ref, b1_ref, rs_ref):
    dinv = dinv_ref[...]
    z = (p_ref[0, :N] + p_ref[1, :N] + h1s_ref[...]) * dinv + b1_ref[...]
    rs_ref[...] = jnp.maximum(z, 0.0) * dinv


def _out_body(q_ref, rs_ref, dinv_ref, b2_ref, w2_ref, o_ref):
    t = (q_ref[0, :N] + q_ref[1, :N] + rs_ref[...]) * dinv_ref[...]
    o = jnp.dot(t, w2_ref[...], preferred_element_type=jnp.float32) + b2_ref[...]
    col = lax.broadcasted_iota(jnp.int32, (N, H), 1)
    om = jnp.where(col < OUT, o, -1e30)
    m = jnp.max(om, axis=1, keepdims=True)
    lse = jnp.log(jnp.sum(jnp.exp(om - m), axis=1, keepdims=True)) + m
    o_ref[...] = om - lse


def kernel(x, edge_index, W1, b1, W2, b2):
    ei = edge_index.astype(jnp.int32)
    npd = EPAD - E
    src_pad = (jnp.arange(npd, dtype=jnp.int32) * 97) % N
    dst_pad = N + (jnp.arange(npd, dtype=jnp.int32) % (NPAD - N))
    srcs = jnp.concatenate([ei[0], src_pad]).reshape(NW, K, C)
    dsts = jnp.concatenate([ei[1], dst_pad]).reshape(NW, K, C)
    zeros1 = jnp.zeros((NPAD,), jnp.float32)
    zeros2 = jnp.zeros((NPAD, H), jnp.float32)
    ones_c = jnp.ones((C,), jnp.float32)
    w2p = jnp.zeros((H, H), jnp.float32).at[:, :OUT].set(W2)
    b1r = b1.reshape(1, H)
    b2r = jnp.zeros((1, H), jnp.float32).at[0, :OUT].set(b2)

    degp = _deg_kernel(dsts, ones_c, zeros1)

    h1s, dinv = pl.pallas_call(
        _mm1_body,
        out_shape=[jax.ShapeDtypeStruct((N, H), jnp.float32),
                   jax.ShapeDtypeStruct((N, 1), jnp.float32)],
    )(x, W1, degp)

    p = _agg_kernel(h1s, srcs, dsts, zeros2)

    rs = pl.pallas_call(
        _mid_body,
        out_shape=jax.ShapeDtypeStruct((N, H), jnp.float32),
    )(p, h1s, dinv, b1r)

    q = _agg_kernel(rs, srcs, dsts, zeros2)

    o = pl.pallas_call(
        _out_body,
        out_shape=jax.ShapeDtypeStruct((N, H), jnp.float32),
    )(q, rs, dinv, b2r, w2p)

    return o[:, :OUT]


# speedup vs baseline: 64.6383x; 1.0002x over previous
import functools
import jax
import jax.numpy as jnp
from jax import lax
from jax.experimental import pallas as pl
from jax.experimental.pallas import tpu as pltpu
from jax.experimental.pallas import tpu_sc as plsc

N = 10000
E = 320000
D = 128
H = 16
OUT = 7

NC = 2
NS = 16
NW = NC * NS
C = 500
K = E // (NW * C)
NPAD = 10240
RS = NPAD // NS
NB = 4
G = K // NB
DEPTH1 = 8
assert NW * K * C == E and K % NB == 0

_sc_mesh = plsc.VectorSubcoreMesh(
    core_axis_name="c", subcore_axis_name="s", num_cores=NC, num_subcores=NS)
_sc_params = pltpu.CompilerParams(use_tc_tiling_on_sc=False)


@functools.partial(
    pl.kernel,
    out_type=jax.ShapeDtypeStruct((NC, NPAD), jnp.float32),
    mesh=_sc_mesh,
    compiler_params=_sc_params,
    scratch_types=[
        pltpu.VMEM((K, C), jnp.int32),
        pltpu.VMEM((C,), jnp.float32),
        pltpu.VMEM_SHARED((NPAD,), jnp.float32),
        pltpu.SemaphoreType.DMA,
    ],
)
def _deg_kernel(dsts_hbm, ones_c_hbm, ones_s_hbm, zeros_s_hbm, out_hbm,
                dst_v, ones_v, acc_sh, ssem):
    c = lax.axis_index("c")
    s = lax.axis_index("s")
    wid = c * NS + s
    pltpu.sync_copy(dsts_hbm.at[wid], dst_v)
    pltpu.sync_copy(ones_c_hbm, ones_v)

    @pl.when(c == 0)
    def _():
        pltpu.sync_copy(ones_s_hbm, acc_sh.at[pl.ds(s * RS, RS)])

    @pl.when(c == 1)
    def _():
        pltpu.sync_copy(zeros_s_hbm, acc_sh.at[pl.ds(s * RS, RS)])

    plsc.subcore_barrier()

    def step(j, carry):
        pltpu.async_copy(ones_v, acc_sh.at[dst_v.at[j]], ssem, add=True)

        @pl.when(j >= DEPTH1)
        def _():
            pltpu.make_async_copy(ones_v, acc_sh.at[dst_v.at[j]], ssem).wait()

        return carry

    lax.fori_loop(0, K, step, 0)
    for _ in range(min(DEPTH1, K)):
        pltpu.make_async_copy(ones_v, acc_sh.at[dst_v.at[0]], ssem).wait()
    plsc.subcore_barrier()
    pltpu.sync_copy(acc_sh.at[pl.ds(s * RS, RS)],
                    out_hbm.at[c, pl.ds(s * RS, RS)])


@functools.partial(
    pl.kernel,
    out_type=jax.ShapeDtypeStruct((NC, NPAD, H), jnp.float32),
    mesh=_sc_mesh,
    compiler_params=_sc_params,
    scratch_types=[
        pltpu.VMEM((K, C), jnp.int32),
        pltpu.VMEM((K, C), jnp.int32),
        pltpu.VMEM((NB, C, H), jnp.float32),
        pltpu.VMEM_SHARED((NPAD, H), jnp.float32),
        pltpu.SemaphoreType.DMA((NB,)),
        pltpu.SemaphoreType.DMA((NB,)),
    ],
)
def _agg_kernel(table_hbm, srcs_hbm, dsts_hbm, zrows_hbm, out_hbm,
                src_v, dst_v, rows_v, acc_sh, gsem, ssem):
    c = lax.axis_index("c")
    s = lax.axis_index("s")
    wid = c * NS + s
    pltpu.sync_copy(srcs_hbm.at[wid], src_v)
    pltpu.sync_copy(dsts_hbm.at[wid], dst_v)

    @pl.when(c == 0)
    def _():
        pltpu.sync_copy(table_hbm.at[pl.ds(s * RS, RS)],
                        acc_sh.at[pl.ds(s * RS, RS)])

    @pl.when(c == 1)
    def _():
        pltpu.sync_copy(zrows_hbm, acc_sh.at[pl.ds(s * RS, RS)])

    plsc.subcore_barrier()

    for b in range(NB):
        pltpu.async_copy(table_hbm.at[src_v.at[b]], rows_v.at[b], gsem.at[b])

    def outer(g, carry):
        base = g * NB
        for b in range(NB):
            j = base + b
            pltpu.make_async_copy(table_hbm.at[src_v.at[j]], rows_v.at[b],
                                  gsem.at[b]).wait()
            pltpu.async_copy(rows_v.at[b], acc_sh.at[dst_v.at[j]],
                             ssem.at[b], add=True)

        @pl.when(g < G - 1)
        def _():
            for b in range(NB):
                j = base + b
                pltpu.make_async_copy(rows_v.at[b], acc_sh.at[dst_v.at[j]],
                                      ssem.at[b]).wait()
                pltpu.async_copy(table_hbm.at[src_v.at[j + NB]],
                                 rows_v.at[b], gsem.at[b])

        return carry

    lax.fori_loop(0, G, outer, 0)
    for b in range(NB):
        pltpu.make_async_copy(rows_v.at[b], acc_sh.at[dst_v.at[b]],
                              ssem.at[b]).wait()
    plsc.subcore_barrier()
    pltpu.sync_copy(acc_sh.at[pl.ds(s * RS, RS)],
                    out_hbm.at[c, pl.ds(s * RS, RS)])


def _mm1_body(x_ref, w1_ref, degp_ref, h1s_ref, dinv_ref):
    deg = degp_ref[0, :N] + degp_ref[1, :N]
    dinv = lax.rsqrt(deg)[:, None]
    h1s_ref[:N, :] = jnp.dot(x_ref[...], w1_ref[...],
                             preferred_element_type=jnp.float32) * dinv
    h1s_ref[N:, :] = jnp.zeros((NPAD - N, H), jnp.float32)
    dinv_ref[...] = dinv


def _mid_body(p_ref, dinv_ref, b1_ref, rs_ref):
    dinv = dinv_ref[...]
    z = (p_ref[0, :N] + p_ref[1, :N]) * dinv + b1_ref[...]
    rs_ref[:N, :] = jnp.maximum(z, 0.0) * dinv
    rs_ref[N:, :] = jnp.zeros((NPAD - N, H), jnp.float32)


def _out_body(q_ref, dinv_ref, b2_ref, w2_ref, o_ref):
    t = (q_ref[0, :N] + q_ref[1, :N]) * dinv_ref[...]
    o = jnp.dot(t, w2_ref[...], preferred_element_type=jnp.float32) + b2_ref[...]
    m = jnp.max(o, axis=1, keepdims=True)
    lse = jnp.log(jnp.sum(jnp.exp(o - m), axis=1, keepdims=True)) + m
    o_ref[...] = o - lse


def kernel(x, edge_index, W1, b1, W2, b2):
    ei = edge_index.astype(jnp.int32)
    srcs = ei[0].reshape(NW, K, C)
    dsts = ei[1].reshape(NW, K, C)
    ones_c = jnp.ones((C,), jnp.float32)
    ones_s = jnp.ones((RS,), jnp.float32)
    zeros_s = jnp.zeros((RS,), jnp.float32)
    zrows = jnp.zeros((RS, H), jnp.float32)
    b1r = b1.reshape(1, H)
    b2r = b2.reshape(1, OUT)

    degp = _deg_kernel(dsts, ones_c, ones_s, zeros_s)

    h1s, dinv = pl.pallas_call(
        _mm1_body,
        out_shape=[jax.ShapeDtypeStruct((NPAD, H), jnp.float32),
                   jax.ShapeDtypeStruct((N, 1), jnp.float32)],
    )(x, W1, degp)

    p = _agg_kernel(h1s, srcs, dsts, zrows)

    rs = pl.pallas_call(
        _mid_body,
        out_shape=jax.ShapeDtypeStruct((NPAD, H), jnp.float32),
    )(p, dinv, b1r)

    q = _agg_kernel(rs, srcs, dsts, zrows)

    o = pl.pallas_call(
        _out_body,
        out_shape=jax.ShapeDtypeStruct((N, OUT), jnp.float32),
    )(q, dinv, b2r, W2)

    return o
